# Initial kernel scaffold; baseline (speedup 1.0000x reference)
import jax
import jax.numpy as jnp
from jax.experimental import pallas as pl


def kernel(t, y, x, query_y, query_x, query_t, T, X, Y):
    raise NotImplementedError("write your pallas kernel here")


# speedup vs baseline: 17.8082x; 17.8082x over previous
import functools

import jax
import jax.numpy as jnp
from jax import lax
from jax.experimental import pallas as pl
from jax.experimental.pallas import tpu as pltpu
from jax.experimental.pallas import tpu_sc as plsc

H = 480
W = 640
D = 64
KS = 9
R = KS // 2
TL = 1.0

PH = H + 2 * R
PW = W + 2 * R
GROWS = PH * PW
C = 128
NTAP = KS * KS
NTAP_PAD = 96

NC = 2
NS = 16
NW = NC * NS

CH_ROWS = 16
CHPIX = CH_ROWS * PW
NCHUNK = -(-PH // CH_ROWS)
GB = 128


def _embed_tc(t_pad, msk, T):
    NP = t_pad.shape[0]
    BN = 2344

    def body(t_ref, m_ref, tt_ref, o_ref):
        ph = (t_ref[:, :] / TL) * tt_ref[0][None, :]
        m = m_ref[:, :]
        o_ref[:, 0:D] = jnp.cos(ph) * m
        o_ref[:, D:2 * D] = jnp.sin(ph) * m

    return pl.pallas_call(
        body,
        grid=(NP // BN,),
        in_specs=[
            pl.BlockSpec((BN, 1), lambda i: (i, 0)),
            pl.BlockSpec((BN, 1), lambda i: (i, 0)),
            pl.BlockSpec((1, D), lambda i: (0, 0)),
        ],
        out_specs=pl.BlockSpec((BN, C), lambda i: (i, 0)),
        out_shape=jax.ShapeDtypeStruct((NP, C), jnp.float32),
    )(t_pad[:, None], msk[:, None], T)


def _weights_tc(X, Y):
    def body(x_ref, y_ref, o_ref):
        k = lax.broadcasted_iota(jnp.int32, (NTAP_PAD, D), 0)
        ky = k // KS
        kx = k - ky * KS
        ny = (ky - R).astype(jnp.float32) / R
        nx = (kx - R).astype(jnp.float32) / R
        ph = nx * x_ref[0][None, :] + ny * y_ref[0][None, :]
        valid = k < NTAP
        o_ref[:, 0:D] = jnp.where(valid, jnp.cos(ph), 0.0)
        o_ref[:, D:2 * D] = jnp.where(valid, jnp.sin(ph), 0.0)

    return pl.pallas_call(
        body,
        out_shape=jax.ShapeDtypeStruct((NTAP_PAD, C), jnp.float32),
    )(X, Y)


def _boxcount_tc(img):
    def body(i_ref, o_ref):
        rows = i_ref[pl.ds(0, H), :]
        for k in range(1, KS):
            rows = rows + i_ref[pl.ds(k, H), :]
        acc = rows[:, 0:W]
        for k in range(1, KS):
            acc = acc + rows[:, k:k + W]
        o_ref[:, :] = acc

    return pl.pallas_call(
        body,
        out_shape=jax.ShapeDtypeStruct((H, W), jnp.float32),
    )(img)


def _finish_tc(S, cq, qt, T):
    MP = S.shape[0]
    BQ = 1280

    def body(s_ref, c_ref, qt_ref, t_ref, or_ref, oi_ref):
        er = s_ref[:, 0:D]
        ei = s_ref[:, D:2 * D]
        cnt = jnp.clip(c_ref[:, :], 1.0, None)
        ph = (qt_ref[:, :] / TL) * t_ref[0][None, :]
        rr = jnp.cos(ph)
        ri = -jnp.sin(ph)
        or_ref[:, :] = (er * rr - ei * ri) / cnt
        oi_ref[:, :] = (er * ri + ei * rr) / cnt

    return pl.pallas_call(
        body,
        grid=(MP // BQ,),
        in_specs=[
            pl.BlockSpec((BQ, C), lambda i: (i, 0)),
            pl.BlockSpec((BQ, 1), lambda i: (i, 0)),
            pl.BlockSpec((BQ, 1), lambda i: (i, 0)),
            pl.BlockSpec((1, D), lambda i: (0, 0)),
        ],
        out_specs=[
            pl.BlockSpec((BQ, D), lambda i: (i, 0)),
            pl.BlockSpec((BQ, D), lambda i: (i, 0)),
        ],
        out_shape=[
            jax.ShapeDtypeStruct((MP, D), jnp.float32),
            jax.ShapeDtypeStruct((MP, D), jnp.float32),
        ],
    )(S, cq, qt, T)


def _scatter_sc(E, eid_sorted, pidx_sorted, offsets):
    mesh = plsc.VectorSubcoreMesh(core_axis_name="c", subcore_axis_name="s")

    @functools.partial(
        pl.kernel,
        mesh=mesh,
        out_type=jax.ShapeDtypeStruct((GROWS, C), jnp.float32),
        scratch_types=[
            pltpu.VMEM((32,), jnp.int32),
            pltpu.VMEM((GB,), jnp.int32),
            pltpu.VMEM((GB,), jnp.int32),
            pltpu.VMEM((1, GB), jnp.int32),
            pltpu.VMEM((GB, C), jnp.float32),
            pltpu.VMEM((72, C), jnp.float32),
            pltpu.VMEM_SHARED((CHPIX + 16, C), jnp.float32),
            pltpu.SemaphoreType.DMA,
        ],
    )
    def k(e_hbm, eid_hbm, pix_hbm, off_hbm, g_hbm, offs_v, ebuf, pxbuf,
          loc2_v, gbuf, zbuf, acc_s, sem):
        core = lax.axis_index("c")
        sub = lax.axis_index("s")
        zv = jnp.zeros((16,), jnp.float32)

        def zrow(r, _):
            for j in range(C // 16):
                zbuf[r, pl.ds(16 * j, 16)] = zv
            return 0

        lax.fori_loop(0, 72, zrow, 0)
        pltpu.sync_copy(off_hbm, offs_v)
        ove0 = offs_v[pl.ds(0, 16)]
        ove1 = offs_v[pl.ds(16, 16)]

        def oget(i):
            return ove0[i] if i < 16 else ove1[i - 16]

        for ch in range(NCHUNK):

            @pl.when(core == (ch % 2))
            def _chunk():
                for j in range(9):
                    pltpu.sync_copy(
                        zbuf, acc_s.at[pl.ds(sub * PW + 72 * j, 72)])
                plsc.subcore_barrier()
                lo = ch * CHPIX
                s = oget(ch)
                e = oget(ch + 1)
                bstart = lax.shift_right_logical(s, 7)
                bend = lax.shift_right_logical(e + (GB - 1), 7)
                nb_t = lax.shift_right_logical(
                    jnp.maximum(0, bend - bstart - sub) + 15, 4)

                def accum(b, _):
                    j = bstart + sub + lax.shift_left(b, 4)
                    pltpu.sync_copy(eid_hbm.at[pl.ds(j * GB, GB)], ebuf)
                    pltpu.sync_copy(pix_hbm.at[pl.ds(j * GB, GB)], pxbuf)
                    for u in range(GB // 16):
                        pv = pxbuf[pl.ds(16 * u, 16)]
                        inm = (pv >= lo) & (pv < lo + CHPIX)
                        loc2_v[0, pl.ds(16 * u, 16)] = jnp.where(
                            inm, pv - lo, CHPIX)
                    pltpu.async_copy(e_hbm.at[ebuf], gbuf, sem).wait()
                    pltpu.sync_copy(gbuf, acc_s.at[loc2_v.at[0]], add=True)
                    return 0

                lax.fori_loop(0, nb_t, accum, 0)
                plsc.subcore_barrier()
                nvalid = PH - ch * CH_ROWS
                if nvalid >= CH_ROWS:
                    pltpu.sync_copy(
                        acc_s.at[pl.ds(sub * PW, PW)],
                        g_hbm.at[pl.ds(lo + sub * PW, PW)])
                else:
                    @pl.when(sub < nvalid)
                    def _tail_copy():
                        pltpu.sync_copy(
                            acc_s.at[pl.ds(sub * PW, PW)],
                            g_hbm.at[pl.ds(lo + sub * PW, PW)])

    return k(E, eid_sorted, pidx_sorted, offsets)


def _gather_reduce_sc(G, qp0, kw, MP):
    QW = MP // NW
    mesh = plsc.VectorSubcoreMesh(core_axis_name="c", subcore_axis_name="s")

    @functools.partial(
        pl.kernel,
        mesh=mesh,
        out_type=jax.ShapeDtypeStruct((MP, C), jnp.float32),
        scratch_types=[
            pltpu.VMEM((NTAP_PAD,), jnp.int32),
            pltpu.VMEM((NTAP_PAD,), jnp.int32),
            pltpu.VMEM((NTAP_PAD, C), jnp.float32),
            pltpu.VMEM((NTAP_PAD, C), jnp.float32),
            pltpu.VMEM((QW,), jnp.int32),
            pltpu.VMEM((QW, C), jnp.float32),
            pltpu.SemaphoreType.DMA,
        ],
    )
    def k(g_hbm, qp0_hbm, kw_hbm, out_hbm, off_v, idx_v, win_v, kw_v, q_v,
          o_v, sem):
        wid = lax.axis_index("s") * NC + lax.axis_index("c")
        base = wid * QW
        pltpu.sync_copy(qp0_hbm.at[pl.ds(base, QW)], q_v)
        pltpu.sync_copy(kw_hbm, kw_v)
        lanes = lax.iota(jnp.int32, 16)
        for j in range(NTAP_PAD // 16):
            kk = lanes + (16 * j)
            ky = lax.shift_right_logical(kk * 57, 9)
            kx = kk - ky * KS
            off = ky * PW + kx
            off = jnp.where(kk < NTAP, off, kk - (NTAP - 1))
            off_v[pl.ds(16 * j, 16)] = off

        def _one_query(i, p0):
            for j in range(NTAP_PAD // 16):
                idx_v[pl.ds(16 * j, 16)] = off_v[pl.ds(16 * j, 16)] + p0
            pltpu.async_copy(g_hbm.at[idx_v], win_v, sem).wait()

            def tap(kt, acc):
                a0, a1, a2, a3, b0, b1, b2, b3 = acc
                gr0 = win_v[kt, pl.ds(0, 16)]
                gr1 = win_v[kt, pl.ds(16, 16)]
                gr2 = win_v[kt, pl.ds(32, 16)]
                gr3 = win_v[kt, pl.ds(48, 16)]
                gi0 = win_v[kt, pl.ds(64, 16)]
                gi1 = win_v[kt, pl.ds(80, 16)]
                gi2 = win_v[kt, pl.ds(96, 16)]
                gi3 = win_v[kt, pl.ds(112, 16)]
                wr0 = kw_v[kt, pl.ds(0, 16)]
                wr1 = kw_v[kt, pl.ds(16, 16)]
                wr2 = kw_v[kt, pl.ds(32, 16)]
                wr3 = kw_v[kt, pl.ds(48, 16)]
                wi0 = kw_v[kt, pl.ds(64, 16)]
                wi1 = kw_v[kt, pl.ds(80, 16)]
                wi2 = kw_v[kt, pl.ds(96, 16)]
                wi3 = kw_v[kt, pl.ds(112, 16)]
                a0 = a0 + gr0 * wr0 - gi0 * wi0
                a1 = a1 + gr1 * wr1 - gi1 * wi1
                a2 = a2 + gr2 * wr2 - gi2 * wi2
                a3 = a3 + gr3 * wr3 - gi3 * wi3
                b0 = b0 + gr0 * wi0 + gi0 * wr0
                b1 = b1 + gr1 * wi1 + gi1 * wr1
                b2 = b2 + gr2 * wi2 + gi2 * wr2
                b3 = b3 + gr3 * wi3 + gi3 * wr3
                return (a0, a1, a2, a3, b0, b1, b2, b3)

            z = jnp.zeros((16,), jnp.float32)
            acc = lax.fori_loop(0, NTAP, tap, (z, z, z, z, z, z, z, z))
            a0, a1, a2, a3, b0, b1, b2, b3 = acc
            o_v[i, pl.ds(0, 16)] = a0
            o_v[i, pl.ds(16, 16)] = a1
            o_v[i, pl.ds(32, 16)] = a2
            o_v[i, pl.ds(48, 16)] = a3
            o_v[i, pl.ds(64, 16)] = b0
            o_v[i, pl.ds(80, 16)] = b1
            o_v[i, pl.ds(96, 16)] = b2
            o_v[i, pl.ds(112, 16)] = b3

        def per_chunk(cq, _):
            chunk = lax.shift_left(cq, 4)
            qvec = q_v[pl.ds(chunk, 16)]
            for lane in range(16):
                _one_query(chunk + lane, qvec[lane])
            return 0

        lax.fori_loop(0, QW // 16, per_chunk, 0)
        pltpu.sync_copy(o_v, out_hbm.at[pl.ds(base, QW)])

    return k(G, qp0, kw)


def kernel(t, y, x, query_y, query_x, query_t, T, X, Y):
    N = t.shape[0]
    M = query_y.shape[0]
    NP = ((N + GB - 1) // GB) * GB
    MP = ((M + 16 * NW - 1) // (16 * NW)) * (16 * NW)

    t_pad = jnp.concatenate([t, jnp.zeros((NP - N,), jnp.float32)])
    msk = jnp.concatenate(
        [jnp.ones((N,), jnp.float32), jnp.zeros((NP - N,), jnp.float32)])
    E = _embed_tc(t_pad, msk, T)

    pidx0 = (y + R) * PW + (x + R)
    pad_pidx = (jnp.arange(NP - N, dtype=jnp.int32) * 991) % GROWS
    pidx = jnp.concatenate([pidx0, pad_pidx])
    cid = pidx // CHPIX
    eid_sorted = jnp.argsort(cid).astype(jnp.int32)
    pidx_sorted = pidx[eid_sorted]
    counts = jnp.bincount(cid, length=NCHUNK)
    offsets = jnp.concatenate(
        [jnp.zeros((1,), jnp.int32),
         jnp.cumsum(counts).astype(jnp.int32)])
    G = _scatter_sc(E, eid_sorted, pidx_sorted, offsets)

    cnt_img = jnp.zeros((GROWS,), jnp.float32).at[pidx0].add(1.0)
    box = _boxcount_tc(cnt_img.reshape(PH, PW))
    cq = box[query_y, query_x]
    cq = jnp.concatenate([cq, jnp.ones((MP - M,), jnp.float32)])[:, None]

    qp0 = query_y * PW + query_x
    pad_p0 = (jnp.arange(MP - M, dtype=jnp.int32) * 337) % (H * PW)
    qp0 = jnp.concatenate([qp0, pad_p0])
    kw = _weights_tc(X, Y)
    S = _gather_reduce_sc(G, qp0, kw, MP)

    qt = jnp.concatenate([query_t, jnp.zeros((MP - M,), jnp.float32)])[:, None]
    outr, outi = _finish_tc(S, cq, qt, T)
    return (outr[:M] + 1j * outi[:M]).astype(jnp.complex64)


# speedup vs baseline: 23.8797x; 1.3409x over previous
import functools

import jax
import jax.numpy as jnp
from jax import lax
from jax.experimental import pallas as pl
from jax.experimental.pallas import tpu as pltpu
from jax.experimental.pallas import tpu_sc as plsc

H = 480
W = 640
D = 64
KS = 9
R = KS // 2
TL = 1.0

PH = H + 2 * R
PW = W + 2 * R
GROWS = PH * PW
C = 128
NTAP = KS * KS
NTAP_PAD = 96

NC = 2
NS = 16
NW = NC * NS

CH_ROWS = 16
CHPIX = CH_ROWS * PW
NCHUNK = -(-PH // CH_ROWS)
GB = 128


def _embed_tc(t_pad, msk, T):
    NP = t_pad.shape[0]
    BN = 2344

    def body(t_ref, m_ref, tt_ref, o_ref):
        ph = (t_ref[:, :] / TL) * tt_ref[0][None, :]
        m = m_ref[:, :]
        o_ref[:, 0:D] = jnp.cos(ph) * m
        o_ref[:, D:2 * D] = jnp.sin(ph) * m

    return pl.pallas_call(
        body,
        grid=(NP // BN,),
        in_specs=[
            pl.BlockSpec((BN, 1), lambda i: (i, 0)),
            pl.BlockSpec((BN, 1), lambda i: (i, 0)),
            pl.BlockSpec((1, D), lambda i: (0, 0)),
        ],
        out_specs=pl.BlockSpec((BN, C), lambda i: (i, 0)),
        out_shape=jax.ShapeDtypeStruct((NP, C), jnp.float32),
    )(t_pad[:, None], msk[:, None], T)


def _weights_tc(X, Y):
    def body(x_ref, y_ref, o_ref):
        k = lax.broadcasted_iota(jnp.int32, (NTAP_PAD, D), 0)
        ky = k // KS
        kx = k - ky * KS
        ny = (ky - R).astype(jnp.float32) / R
        nx = (kx - R).astype(jnp.float32) / R
        ph = nx * x_ref[0][None, :] + ny * y_ref[0][None, :]
        valid = k < NTAP
        o_ref[:, 0:D] = jnp.where(valid, jnp.cos(ph), 0.0)
        o_ref[:, D:2 * D] = jnp.where(valid, jnp.sin(ph), 0.0)

    return pl.pallas_call(
        body,
        out_shape=jax.ShapeDtypeStruct((NTAP_PAD, C), jnp.float32),
    )(X, Y)


def _boxcount_tc(img):
    def body(i_ref, o_ref):
        rows = i_ref[pl.ds(0, H), :]
        for k in range(1, KS):
            rows = rows + i_ref[pl.ds(k, H), :]
        acc = rows[:, 0:W]
        for k in range(1, KS):
            acc = acc + rows[:, k:k + W]
        o_ref[:, :] = acc

    return pl.pallas_call(
        body,
        out_shape=jax.ShapeDtypeStruct((H, W), jnp.float32),
    )(img)


def _finish_tc(S, cq, qt, T):
    MP = S.shape[0]
    BQ = 1280

    def body(s_ref, c_ref, qt_ref, t_ref, or_ref, oi_ref):
        er = s_ref[:, 0:D]
        ei = s_ref[:, D:2 * D]
        cnt = jnp.clip(c_ref[:, :], 1.0, None)
        ph = (qt_ref[:, :] / TL) * t_ref[0][None, :]
        rr = jnp.cos(ph)
        ri = -jnp.sin(ph)
        or_ref[:, :] = (er * rr - ei * ri) / cnt
        oi_ref[:, :] = (er * ri + ei * rr) / cnt

    return pl.pallas_call(
        body,
        grid=(MP // BQ,),
        in_specs=[
            pl.BlockSpec((BQ, C), lambda i: (i, 0)),
            pl.BlockSpec((BQ, 1), lambda i: (i, 0)),
            pl.BlockSpec((BQ, 1), lambda i: (i, 0)),
            pl.BlockSpec((1, D), lambda i: (0, 0)),
        ],
        out_specs=[
            pl.BlockSpec((BQ, D), lambda i: (i, 0)),
            pl.BlockSpec((BQ, D), lambda i: (i, 0)),
        ],
        out_shape=[
            jax.ShapeDtypeStruct((MP, D), jnp.float32),
            jax.ShapeDtypeStruct((MP, D), jnp.float32),
        ],
    )(S, cq, qt, T)


CROWS = CHPIX // C
CROWS_PAD = 88


def _scatter_sc(E, eid_sorted, pidx_sorted, offsets):
    mesh = plsc.VectorSubcoreMesh(core_axis_name="c", subcore_axis_name="s")

    @functools.partial(
        pl.kernel,
        mesh=mesh,
        out_type=[
            jax.ShapeDtypeStruct((GROWS, C), jnp.float32),
            jax.ShapeDtypeStruct((NCHUNK * CHPIX,), jnp.float32),
        ],
        scratch_types=[
            pltpu.VMEM((32,), jnp.int32),
            pltpu.VMEM((GB,), jnp.int32),
            pltpu.VMEM((GB,), jnp.int32),
            pltpu.VMEM((1, GB), jnp.int32),
            pltpu.VMEM((GB,), jnp.int32),
            pltpu.VMEM((GB,), jnp.float32),
            pltpu.VMEM((GB, C), jnp.float32),
            pltpu.VMEM((72, C), jnp.float32),
            pltpu.VMEM((2624,), jnp.float32),
            pltpu.VMEM_SHARED((CHPIX + 16, C), jnp.float32),
            pltpu.VMEM_SHARED((CHPIX + GB, ), jnp.float32),
            pltpu.SemaphoreType.DMA,
        ],
    )
    def k(e_hbm, eid_hbm, pix_hbm, off_hbm, g_hbm, gc_hbm, offs_v, ebuf,
          pxbuf, loc2_v, loc1_v, ones_v, gbuf, zbuf, z1_v, acc_s, accc_s,
          sem):
        core = lax.axis_index("c")
        sub = lax.axis_index("s")
        zv = jnp.zeros((16,), jnp.float32)
        lanes16 = lax.iota(jnp.int32, 16)

        def zrow(r, _):
            for j in range(C // 16):
                zbuf[r, pl.ds(16 * j, 16)] = zv
            return 0

        lax.fori_loop(0, 72, zrow, 0)

        def zrow2(r, _):
            z1_v[pl.ds(r * 16, 16)] = zv
            return 0

        lax.fori_loop(0, 164, zrow2, 0)
        ov = jnp.full((16,), 1.0, jnp.float32)
        for j in range(GB // 16):
            ones_v[pl.ds(16 * j, 16)] = ov
        pltpu.sync_copy(off_hbm, offs_v)
        ove0 = offs_v[pl.ds(0, 16)]
        ove1 = offs_v[pl.ds(16, 16)]

        def oget(i):
            return ove0[i] if i < 16 else ove1[i - 16]

        for ch in range(NCHUNK):

            @pl.when(core == (ch % 2))
            def _chunk():
                for j in range(9):
                    pltpu.sync_copy(
                        zbuf, acc_s.at[pl.ds(sub * PW + 72 * j, 72)])

                @pl.when(sub < 4)
                def _zero_counts():
                    pltpu.sync_copy(z1_v,
                                    accc_s.at[pl.ds(sub * 2624, 2624)])

                plsc.subcore_barrier()
                lo = ch * CHPIX
                s = oget(ch)
                e = oget(ch + 1)
                bstart = lax.shift_right_logical(s, 7)
                bend = lax.shift_right_logical(e + (GB - 1), 7)
                nb_t = lax.shift_right_logical(
                    jnp.maximum(0, bend - bstart - sub) + 15, 4)

                def accum(b, _):
                    j = bstart + sub + lax.shift_left(b, 4)
                    pltpu.sync_copy(eid_hbm.at[pl.ds(j * GB, GB)], ebuf)
                    pltpu.sync_copy(pix_hbm.at[pl.ds(j * GB, GB)], pxbuf)
                    for u in range(GB // 16):
                        pv = pxbuf[pl.ds(16 * u, 16)]
                        inm = (pv >= lo) & (pv < lo + CHPIX)
                        loc = jnp.where(inm, pv - lo, CHPIX)
                        loc2_v[0, pl.ds(16 * u, 16)] = loc
                        loc1_v[pl.ds(16 * u, 16)] = jnp.where(
                            inm, pv - lo, CHPIX + lanes16 + 16 * u)
                    pltpu.async_copy(e_hbm.at[ebuf], gbuf, sem).wait()
                    pltpu.sync_copy(gbuf, acc_s.at[loc2_v.at[0]], add=True)
                    pltpu.sync_copy(ones_v, accc_s.at[loc1_v], add=True)
                    return 0

                lax.fori_loop(0, nb_t, accum, 0)
                plsc.subcore_barrier()

                @pl.when(sub == 0)
                def _copy_counts():
                    pltpu.sync_copy(
                        accc_s.at[pl.ds(0, CHPIX)],
                        gc_hbm.at[pl.ds(ch * CHPIX, CHPIX)])

                nvalid = PH - ch * CH_ROWS
                if nvalid >= CH_ROWS:
                    pltpu.sync_copy(
                        acc_s.at[pl.ds(sub * PW, PW)],
                        g_hbm.at[pl.ds(lo + sub * PW, PW)])
                else:
                    @pl.when(sub < nvalid)
                    def _tail_copy():
                        pltpu.sync_copy(
                            acc_s.at[pl.ds(sub * PW, PW)],
                            g_hbm.at[pl.ds(lo + sub * PW, PW)])

    return k(E, eid_sorted, pidx_sorted, offsets)


def _gather_reduce_sc(G, qp0, kw, MP):
    QW = MP // NW
    mesh = plsc.VectorSubcoreMesh(core_axis_name="c", subcore_axis_name="s")

    @functools.partial(
        pl.kernel,
        mesh=mesh,
        out_type=jax.ShapeDtypeStruct((MP, C), jnp.float32),
        scratch_types=[
            pltpu.VMEM((NTAP_PAD,), jnp.int32),
            pltpu.VMEM((NTAP_PAD,), jnp.int32),
            pltpu.VMEM((NTAP_PAD, C), jnp.float32),
            pltpu.VMEM((NTAP_PAD, C), jnp.float32),
            pltpu.VMEM((QW,), jnp.int32),
            pltpu.VMEM((QW, C), jnp.float32),
            pltpu.SemaphoreType.DMA,
        ],
    )
    def k(g_hbm, qp0_hbm, kw_hbm, out_hbm, off_v, idx_v, win_v, kw_v, q_v,
          o_v, sem):
        wid = lax.axis_index("s") * NC + lax.axis_index("c")
        base = wid * QW
        pltpu.sync_copy(qp0_hbm.at[pl.ds(base, QW)], q_v)
        pltpu.sync_copy(kw_hbm, kw_v)
        lanes = lax.iota(jnp.int32, 16)
        for j in range(NTAP_PAD // 16):
            kk = lanes + (16 * j)
            ky = lax.shift_right_logical(kk * 57, 9)
            kx = kk - ky * KS
            off = ky * PW + kx
            off = jnp.where(kk < NTAP, off, kk - (NTAP - 1))
            off_v[pl.ds(16 * j, 16)] = off

        def _one_query(i, p0):
            for j in range(NTAP_PAD // 16):
                idx_v[pl.ds(16 * j, 16)] = off_v[pl.ds(16 * j, 16)] + p0
            pltpu.async_copy(g_hbm.at[idx_v], win_v, sem).wait()

            def tap(kt, acc):
                a0, a1, a2, a3, b0, b1, b2, b3 = acc
                gr0 = win_v[kt, pl.ds(0, 16)]
                gr1 = win_v[kt, pl.ds(16, 16)]
                gr2 = win_v[kt, pl.ds(32, 16)]
                gr3 = win_v[kt, pl.ds(48, 16)]
                gi0 = win_v[kt, pl.ds(64, 16)]
                gi1 = win_v[kt, pl.ds(80, 16)]
                gi2 = win_v[kt, pl.ds(96, 16)]
                gi3 = win_v[kt, pl.ds(112, 16)]
                wr0 = kw_v[kt, pl.ds(0, 16)]
                wr1 = kw_v[kt, pl.ds(16, 16)]
                wr2 = kw_v[kt, pl.ds(32, 16)]
                wr3 = kw_v[kt, pl.ds(48, 16)]
                wi0 = kw_v[kt, pl.ds(64, 16)]
                wi1 = kw_v[kt, pl.ds(80, 16)]
                wi2 = kw_v[kt, pl.ds(96, 16)]
                wi3 = kw_v[kt, pl.ds(112, 16)]
                a0 = a0 + gr0 * wr0 - gi0 * wi0
                a1 = a1 + gr1 * wr1 - gi1 * wi1
                a2 = a2 + gr2 * wr2 - gi2 * wi2
                a3 = a3 + gr3 * wr3 - gi3 * wi3
                b0 = b0 + gr0 * wi0 + gi0 * wr0
                b1 = b1 + gr1 * wi1 + gi1 * wr1
                b2 = b2 + gr2 * wi2 + gi2 * wr2
                b3 = b3 + gr3 * wi3 + gi3 * wr3
                return (a0, a1, a2, a3, b0, b1, b2, b3)

            z = jnp.zeros((16,), jnp.float32)
            acc = lax.fori_loop(0, NTAP, tap, (z, z, z, z, z, z, z, z))
            a0, a1, a2, a3, b0, b1, b2, b3 = acc
            o_v[i, pl.ds(0, 16)] = a0
            o_v[i, pl.ds(16, 16)] = a1
            o_v[i, pl.ds(32, 16)] = a2
            o_v[i, pl.ds(48, 16)] = a3
            o_v[i, pl.ds(64, 16)] = b0
            o_v[i, pl.ds(80, 16)] = b1
            o_v[i, pl.ds(96, 16)] = b2
            o_v[i, pl.ds(112, 16)] = b3

        def per_chunk(cq, _):
            chunk = lax.shift_left(cq, 4)
            qvec = q_v[pl.ds(chunk, 16)]
            for lane in range(16):
                _one_query(chunk + lane, qvec[lane])
            return 0

        lax.fori_loop(0, QW // 16, per_chunk, 0)
        pltpu.sync_copy(o_v, out_hbm.at[pl.ds(base, QW)])

    return k(G, qp0, kw)


def kernel(t, y, x, query_y, query_x, query_t, T, X, Y):
    N = t.shape[0]
    M = query_y.shape[0]
    NP = ((N + GB - 1) // GB) * GB
    MP = ((M + 16 * NW - 1) // (16 * NW)) * (16 * NW)

    t_pad = jnp.concatenate([t, jnp.zeros((NP - N,), jnp.float32)])
    msk = jnp.concatenate(
        [jnp.ones((N,), jnp.float32), jnp.zeros((NP - N,), jnp.float32)])
    E = _embed_tc(t_pad, msk, T)

    pidx0 = (y + R) * PW + (x + R)
    pad_pidx = jnp.full((NP - N,), NCHUNK * CHPIX - 1, jnp.int32)
    pidx = jnp.concatenate([pidx0, pad_pidx])
    cid = pidx // CHPIX
    eid_sorted = jnp.argsort(cid).astype(jnp.int32)
    pidx_sorted = pidx[eid_sorted]
    counts = jnp.bincount(cid, length=NCHUNK)
    offsets = jnp.concatenate(
        [jnp.zeros((1,), jnp.int32),
         jnp.cumsum(counts).astype(jnp.int32)])
    G, Gc = _scatter_sc(E, eid_sorted, pidx_sorted, offsets)

    cnt_img = Gc[:GROWS]
    box = _boxcount_tc(cnt_img.reshape(PH, PW))
    cq = box[query_y, query_x]
    cq = jnp.concatenate([cq, jnp.ones((MP - M,), jnp.float32)])[:, None]

    qp0 = query_y * PW + query_x
    pad_p0 = (jnp.arange(MP - M, dtype=jnp.int32) * 337) % (H * PW)
    qp0 = jnp.concatenate([qp0, pad_p0])
    kw = _weights_tc(X, Y)
    S = _gather_reduce_sc(G, qp0, kw, MP)

    qt = jnp.concatenate([query_t, jnp.zeros((MP - M,), jnp.float32)])[:, None]
    outr, outi = _finish_tc(S, cq, qt, T)
    return (outr[:M] + 1j * outi[:M]).astype(jnp.complex64)


# speedup vs baseline: 27.2360x; 1.1405x over previous
import functools

import jax
import jax.numpy as jnp
from jax import lax
from jax.experimental import pallas as pl
from jax.experimental.pallas import tpu as pltpu
from jax.experimental.pallas import tpu_sc as plsc

H = 480
W = 640
D = 64
KS = 9
R = KS // 2
TL = 1.0

PH = H + 2 * R
PW = W + 2 * R
GROWS = PH * PW
C = 128
NTAP = KS * KS
NTAP_PAD = 96

NC = 2
NS = 16
NW = NC * NS

CH_ROWS = 16
CHPIX = CH_ROWS * PW
NCHUNK = -(-PH // CH_ROWS)
GB = 128


def _embed_tc(t_pad, msk, T):
    NP = t_pad.shape[0]
    BN = 2344

    def body(t_ref, m_ref, tt_ref, o_ref):
        ph = (t_ref[:, :] / TL) * tt_ref[0][None, :]
        m = m_ref[:, :]
        o_ref[:, 0:D] = jnp.cos(ph) * m
        o_ref[:, D:2 * D] = jnp.sin(ph) * m

    return pl.pallas_call(
        body,
        grid=(NP // BN,),
        in_specs=[
            pl.BlockSpec((BN, 1), lambda i: (i, 0)),
            pl.BlockSpec((BN, 1), lambda i: (i, 0)),
            pl.BlockSpec((1, D), lambda i: (0, 0)),
        ],
        out_specs=pl.BlockSpec((BN, C), lambda i: (i, 0)),
        out_shape=jax.ShapeDtypeStruct((NP, C), jnp.float32),
    )(t_pad[:, None], msk[:, None], T)


def _weights_tc(X, Y):
    def body(x_ref, y_ref, o_ref):
        k = lax.broadcasted_iota(jnp.int32, (NTAP_PAD, D), 0)
        ky = k // KS
        kx = k - ky * KS
        ny = (ky - R).astype(jnp.float32) / R
        nx = (kx - R).astype(jnp.float32) / R
        ph = nx * x_ref[0][None, :] + ny * y_ref[0][None, :]
        valid = k < NTAP
        o_ref[:, 0:D] = jnp.where(valid, jnp.cos(ph), 0.0)
        o_ref[:, D:2 * D] = jnp.where(valid, jnp.sin(ph), 0.0)

    return pl.pallas_call(
        body,
        out_shape=jax.ShapeDtypeStruct((NTAP_PAD, C), jnp.float32),
    )(X, Y)


def _boxcount_tc(img):
    def body(i_ref, o_ref):
        rows = i_ref[pl.ds(0, H), :]
        for k in range(1, KS):
            rows = rows + i_ref[pl.ds(k, H), :]
        acc = rows[:, 0:W]
        for k in range(1, KS):
            acc = acc + rows[:, k:k + W]
        o_ref[:, :] = acc

    return pl.pallas_call(
        body,
        out_shape=jax.ShapeDtypeStruct((H, W), jnp.float32),
    )(img)


def _finish_tc(S, cq, qt, T):
    MP = S.shape[0]
    BQ = 1280

    def body(s_ref, c_ref, qt_ref, t_ref, or_ref, oi_ref):
        er = s_ref[:, 0:D]
        ei = s_ref[:, D:2 * D]
        cnt = jnp.clip(c_ref[:, :], 1.0, None)
        ph = (qt_ref[:, :] / TL) * t_ref[0][None, :]
        rr = jnp.cos(ph)
        ri = -jnp.sin(ph)
        or_ref[:, :] = (er * rr - ei * ri) / cnt
        oi_ref[:, :] = (er * ri + ei * rr) / cnt

    return pl.pallas_call(
        body,
        grid=(MP // BQ,),
        in_specs=[
            pl.BlockSpec((BQ, C), lambda i: (i, 0)),
            pl.BlockSpec((BQ, 1), lambda i: (i, 0)),
            pl.BlockSpec((BQ, 1), lambda i: (i, 0)),
            pl.BlockSpec((1, D), lambda i: (0, 0)),
        ],
        out_specs=[
            pl.BlockSpec((BQ, D), lambda i: (i, 0)),
            pl.BlockSpec((BQ, D), lambda i: (i, 0)),
        ],
        out_shape=[
            jax.ShapeDtypeStruct((MP, D), jnp.float32),
            jax.ShapeDtypeStruct((MP, D), jnp.float32),
        ],
    )(S, cq, qt, T)


CROWS = CHPIX // C
CROWS_PAD = 88


def _scatter_sc(E, eid_sorted, pidx_sorted, offsets):
    mesh = plsc.VectorSubcoreMesh(core_axis_name="c", subcore_axis_name="s")

    @functools.partial(
        pl.kernel,
        mesh=mesh,
        out_type=[
            jax.ShapeDtypeStruct((GROWS, C), jnp.float32),
            jax.ShapeDtypeStruct((NCHUNK * CHPIX,), jnp.float32),
        ],
        scratch_types=[
            pltpu.VMEM((32,), jnp.int32),
            pltpu.VMEM((GB,), jnp.int32),
            pltpu.VMEM((GB,), jnp.int32),
            pltpu.VMEM((1, GB), jnp.int32),
            pltpu.VMEM((GB,), jnp.int32),
            pltpu.VMEM((GB,), jnp.float32),
            pltpu.VMEM((GB, C), jnp.float32),
            pltpu.VMEM((72, C), jnp.float32),
            pltpu.VMEM((2624,), jnp.float32),
            pltpu.VMEM_SHARED((CHPIX + 16, C), jnp.float32),
            pltpu.VMEM_SHARED((CHPIX + GB, ), jnp.float32),
            pltpu.SemaphoreType.DMA,
        ],
    )
    def k(e_hbm, eid_hbm, pix_hbm, off_hbm, g_hbm, gc_hbm, offs_v, ebuf,
          pxbuf, loc2_v, loc1_v, ones_v, gbuf, zbuf, z1_v, acc_s, accc_s,
          sem):
        core = lax.axis_index("c")
        sub = lax.axis_index("s")
        zv = jnp.zeros((16,), jnp.float32)
        lanes16 = lax.iota(jnp.int32, 16)

        def zrow(r, _):
            for j in range(C // 16):
                zbuf[r, pl.ds(16 * j, 16)] = zv
            return 0

        lax.fori_loop(0, 72, zrow, 0)

        def zrow2(r, _):
            z1_v[pl.ds(r * 16, 16)] = zv
            return 0

        lax.fori_loop(0, 164, zrow2, 0)
        ov = jnp.full((16,), 1.0, jnp.float32)
        for j in range(GB // 16):
            ones_v[pl.ds(16 * j, 16)] = ov
        pltpu.sync_copy(off_hbm, offs_v)
        ove0 = offs_v[pl.ds(0, 16)]
        ove1 = offs_v[pl.ds(16, 16)]

        def oget(i):
            return ove0[i] if i < 16 else ove1[i - 16]

        for ch in range(NCHUNK):

            @pl.when(core == (ch % 2))
            def _chunk():
                for j in range(9):
                    pltpu.sync_copy(
                        zbuf, acc_s.at[pl.ds(sub * PW + 72 * j, 72)])

                @pl.when(sub < 4)
                def _zero_counts():
                    pltpu.sync_copy(z1_v,
                                    accc_s.at[pl.ds(sub * 2624, 2624)])

                plsc.subcore_barrier()
                lo = ch * CHPIX
                s = oget(ch)
                e = oget(ch + 1)
                bstart = lax.shift_right_logical(s, 7)
                bend = lax.shift_right_logical(e + (GB - 1), 7)
                nb_t = lax.shift_right_logical(
                    jnp.maximum(0, bend - bstart - sub) + 15, 4)

                def accum(b, _):
                    j = bstart + sub + lax.shift_left(b, 4)
                    pltpu.sync_copy(eid_hbm.at[pl.ds(j * GB, GB)], ebuf)
                    pltpu.sync_copy(pix_hbm.at[pl.ds(j * GB, GB)], pxbuf)
                    for u in range(GB // 16):
                        pv = pxbuf[pl.ds(16 * u, 16)]
                        inm = (pv >= lo) & (pv < lo + CHPIX)
                        loc = jnp.where(inm, pv - lo, CHPIX)
                        loc2_v[0, pl.ds(16 * u, 16)] = loc
                        loc1_v[pl.ds(16 * u, 16)] = jnp.where(
                            inm, pv - lo, CHPIX + lanes16 + 16 * u)
                    pltpu.async_copy(e_hbm.at[ebuf], gbuf, sem).wait()
                    pltpu.sync_copy(gbuf, acc_s.at[loc2_v.at[0]], add=True)
                    pltpu.sync_copy(ones_v, accc_s.at[loc1_v], add=True)
                    return 0

                lax.fori_loop(0, nb_t, accum, 0)
                plsc.subcore_barrier()

                @pl.when(sub == 0)
                def _copy_counts():
                    pltpu.sync_copy(
                        accc_s.at[pl.ds(0, CHPIX)],
                        gc_hbm.at[pl.ds(ch * CHPIX, CHPIX)])

                nvalid = PH - ch * CH_ROWS
                if nvalid >= CH_ROWS:
                    pltpu.sync_copy(
                        acc_s.at[pl.ds(sub * PW, PW)],
                        g_hbm.at[pl.ds(lo + sub * PW, PW)])
                else:
                    @pl.when(sub < nvalid)
                    def _tail_copy():
                        pltpu.sync_copy(
                            acc_s.at[pl.ds(sub * PW, PW)],
                            g_hbm.at[pl.ds(lo + sub * PW, PW)])

    return k(E, eid_sorted, pidx_sorted, offsets)


def _gather_reduce_sc(G, qp0, kw, MP):
    QW = MP // NW
    mesh = plsc.VectorSubcoreMesh(core_axis_name="c", subcore_axis_name="s")

    @functools.partial(
        pl.kernel,
        mesh=mesh,
        out_type=jax.ShapeDtypeStruct((MP, C), jnp.float32),
        scratch_types=[
            pltpu.VMEM((NTAP_PAD,), jnp.int32),
            pltpu.VMEM((NTAP_PAD,), jnp.int32),
            pltpu.VMEM((NTAP_PAD,), jnp.int32),
            pltpu.VMEM((NTAP_PAD, C), jnp.float32),
            pltpu.VMEM((NTAP_PAD, C), jnp.float32),
            pltpu.VMEM((NTAP_PAD, C), jnp.float32),
            pltpu.VMEM((QW,), jnp.int32),
            pltpu.VMEM((QW, C), jnp.float32),
            pltpu.SemaphoreType.DMA,
            pltpu.SemaphoreType.DMA,
        ],
    )
    def k(g_hbm, qp0_hbm, kw_hbm, out_hbm, off_v, idx_va, idx_vb, win_va,
          win_vb, kw_v, q_v, o_v, sema, semb):
        wid = lax.axis_index("s") * NC + lax.axis_index("c")
        base = wid * QW
        pltpu.sync_copy(qp0_hbm.at[pl.ds(base, QW)], q_v)
        pltpu.sync_copy(kw_hbm, kw_v)
        lanes = lax.iota(jnp.int32, 16)
        for j in range(NTAP_PAD // 16):
            kk = lanes + (16 * j)
            ky = lax.shift_right_logical(kk * 57, 9)
            kx = kk - ky * KS
            off = ky * PW + kx
            off = jnp.where(kk < NTAP, off, kk - (NTAP - 1))
            off_v[pl.ds(16 * j, 16)] = off

        idxs = (idx_va, idx_vb)
        wins = (win_va, win_vb)
        sems = (sema, semb)

        def _issue(p0, b):
            for j in range(NTAP_PAD // 16):
                idxs[b][pl.ds(16 * j, 16)] = off_v[pl.ds(16 * j, 16)] + p0
            return pltpu.async_copy(g_hbm.at[idxs[b]], wins[b], sems[b])

        def _compute(i, win_v):

            def tap(kt, acc):
                a0, a1, a2, a3, b0, b1, b2, b3 = acc
                gr0 = win_v[kt, pl.ds(0, 16)]
                gr1 = win_v[kt, pl.ds(16, 16)]
                gr2 = win_v[kt, pl.ds(32, 16)]
                gr3 = win_v[kt, pl.ds(48, 16)]
                gi0 = win_v[kt, pl.ds(64, 16)]
                gi1 = win_v[kt, pl.ds(80, 16)]
                gi2 = win_v[kt, pl.ds(96, 16)]
                gi3 = win_v[kt, pl.ds(112, 16)]
                wr0 = kw_v[kt, pl.ds(0, 16)]
                wr1 = kw_v[kt, pl.ds(16, 16)]
                wr2 = kw_v[kt, pl.ds(32, 16)]
                wr3 = kw_v[kt, pl.ds(48, 16)]
                wi0 = kw_v[kt, pl.ds(64, 16)]
                wi1 = kw_v[kt, pl.ds(80, 16)]
                wi2 = kw_v[kt, pl.ds(96, 16)]
                wi3 = kw_v[kt, pl.ds(112, 16)]
                a0 = a0 + gr0 * wr0 - gi0 * wi0
                a1 = a1 + gr1 * wr1 - gi1 * wi1
                a2 = a2 + gr2 * wr2 - gi2 * wi2
                a3 = a3 + gr3 * wr3 - gi3 * wi3
                b0 = b0 + gr0 * wi0 + gi0 * wr0
                b1 = b1 + gr1 * wi1 + gi1 * wr1
                b2 = b2 + gr2 * wi2 + gi2 * wr2
                b3 = b3 + gr3 * wi3 + gi3 * wr3
                return (a0, a1, a2, a3, b0, b1, b2, b3)

            z = jnp.zeros((16,), jnp.float32)
            acc = lax.fori_loop(0, NTAP, tap, (z, z, z, z, z, z, z, z))
            a0, a1, a2, a3, b0, b1, b2, b3 = acc
            o_v[i, pl.ds(0, 16)] = a0
            o_v[i, pl.ds(16, 16)] = a1
            o_v[i, pl.ds(32, 16)] = a2
            o_v[i, pl.ds(48, 16)] = a3
            o_v[i, pl.ds(64, 16)] = b0
            o_v[i, pl.ds(80, 16)] = b1
            o_v[i, pl.ds(96, 16)] = b2
            o_v[i, pl.ds(112, 16)] = b3

        def per_chunk(cq, _):
            chunk = lax.shift_left(cq, 4)
            qvec = q_v[pl.ds(chunk, 16)]
            cp = _issue(qvec[0], 0)
            for lane in range(16):
                nxt = (_issue(qvec[lane + 1], (lane + 1) & 1)
                       if lane < 15 else None)
                cp.wait()
                _compute(chunk + lane, wins[lane & 1])
                cp = nxt
            return 0

        lax.fori_loop(0, QW // 16, per_chunk, 0)
        pltpu.sync_copy(o_v, out_hbm.at[pl.ds(base, QW)])

    return k(G, qp0, kw)


def kernel(t, y, x, query_y, query_x, query_t, T, X, Y):
    N = t.shape[0]
    M = query_y.shape[0]
    NP = ((N + GB - 1) // GB) * GB
    MP = ((M + 16 * NW - 1) // (16 * NW)) * (16 * NW)

    t_pad = jnp.concatenate([t, jnp.zeros((NP - N,), jnp.float32)])
    msk = jnp.concatenate(
        [jnp.ones((N,), jnp.float32), jnp.zeros((NP - N,), jnp.float32)])
    E = _embed_tc(t_pad, msk, T)

    pidx0 = (y + R) * PW + (x + R)
    pad_pidx = jnp.full((NP - N,), NCHUNK * CHPIX - 1, jnp.int32)
    pidx = jnp.concatenate([pidx0, pad_pidx])
    cid = pidx // CHPIX
    eid_sorted = jnp.argsort(cid).astype(jnp.int32)
    pidx_sorted = pidx[eid_sorted]
    counts = jnp.bincount(cid, length=NCHUNK)
    offsets = jnp.concatenate(
        [jnp.zeros((1,), jnp.int32),
         jnp.cumsum(counts).astype(jnp.int32)])
    G, Gc = _scatter_sc(E, eid_sorted, pidx_sorted, offsets)

    cnt_img = Gc[:GROWS]
    box = _boxcount_tc(cnt_img.reshape(PH, PW))
    cq = box[query_y, query_x]
    cq = jnp.concatenate([cq, jnp.ones((MP - M,), jnp.float32)])[:, None]

    qp0 = query_y * PW + query_x
    pad_p0 = (jnp.arange(MP - M, dtype=jnp.int32) * 337) % (H * PW)
    qp0 = jnp.concatenate([qp0, pad_p0])
    kw = _weights_tc(X, Y)
    S = _gather_reduce_sc(G, qp0, kw, MP)

    qt = jnp.concatenate([query_t, jnp.zeros((MP - M,), jnp.float32)])[:, None]
    outr, outi = _finish_tc(S, cq, qt, T)
    return (outr[:M] + 1j * outi[:M]).astype(jnp.complex64)
